# baseline (speedup 1.0000x reference)
import functools

import jax
import jax.numpy as jnp
from jax import lax
from jax.experimental import pallas as pl
from jax.experimental.pallas import tpu as pltpu
import jax.experimental.pallas.tpu_sc as plsc

_E = 100000
_N = 50000
_P = 50000
_FIN = 32
_DE = 16
_CH = 32

_NC = 2
_NS = 16
_NW = _NC * _NS

_CHUNK = 128
_KG = 25
_EPAD = _NW * _KG * _CHUNK
_BE = 1024
_GE = _EPAD // _BE
_BN = 512
_NPAD = _GE * _BN
_KS = ((_EPAD + _NPAD) // _NS) // _CHUNK
_PPW = 1568
_PPAD = _NW * _PPW
_DUMMY = _N

_SC_PARAMS = pltpu.CompilerParams(use_tc_tiling_on_sc=False,
                                  needs_layout_passes=False)


def _sc_gather(table128, idx128):
    kg = idx128.shape[0] // _NW
    bpw = kg * _CHUNK
    mesh = plsc.VectorSubcoreMesh(core_axis_name="c", subcore_axis_name="s")

    @functools.partial(
        pl.kernel,
        out_type=jax.ShapeDtypeStruct((_NW * bpw, 128), jnp.float32),
        mesh=mesh,
        compiler_params=_SC_PARAMS,
        scratch_types=[
            pltpu.VMEM((kg, _CHUNK), jnp.int32),
            pltpu.VMEM((4, _CHUNK, 128), jnp.float32),
            pltpu.SemaphoreType.DMA,
            pltpu.SemaphoreType.DMA,
        ],
    )
    def k(table_hbm, idx_hbm, out_hbm, idx_v, fbuf, fsem, wsem):
        w = lax.axis_index("s") * _NC + lax.axis_index("c")
        pltpu.sync_copy(idx_hbm.at[pl.ds(w * kg, kg)], idx_v)

        def fetch(j):
            pltpu.async_copy(table_hbm.at[idx_v.at[j]],
                             fbuf.at[lax.rem(j, 4)], fsem)

        def fetch_wait(j):
            pltpu.make_async_copy(table_hbm.at[idx_v.at[j]],
                                  fbuf.at[lax.rem(j, 4)], fsem).wait()

        def write(j):
            pltpu.async_copy(
                fbuf.at[lax.rem(j, 4), pl.ds(0, _CHUNK), pl.ds(0, _CH)],
                out_hbm.at[pl.ds(w * bpw + j * _CHUNK, _CHUNK),
                           pl.ds(0, _CH)], wsem)

        def write_wait(j):
            pltpu.make_async_copy(
                fbuf.at[lax.rem(j, 4), pl.ds(0, _CHUNK), pl.ds(0, _CH)],
                out_hbm.at[pl.ds(w * bpw + j * _CHUNK, _CHUNK),
                           pl.ds(0, _CH)], wsem).wait()

        fetch(0)
        fetch(1)

        def step(j, c):
            @pl.when(j >= 2)
            def _():
                write_wait(j - 2)
            fetch_wait(j)

            @pl.when(j + 2 < kg)
            def _():
                fetch(j + 2)
            write(j)
            return c

        lax.fori_loop(0, kg, step, 0)
        write_wait(kg - 2)
        write_wait(kg - 1)

    return k(table128, idx128)


def _sc_scatter(m128, xr128, dstE):
    ke = dstE.shape[0] // _NS
    npad = xr128.shape[0]
    rpt = npad // _NS
    ept = ke * _CHUNK
    mesh = plsc.VectorSubcoreMesh(core_axis_name="c", subcore_axis_name="s",
                                  num_cores=1)

    @functools.partial(
        pl.kernel,
        out_type=jax.ShapeDtypeStruct((npad, 128), jnp.float32),
        mesh=mesh,
        compiler_params=_SC_PARAMS,
        scratch_types=[
            pltpu.VMEM((ke, _CHUNK), jnp.int32),
            pltpu.VMEM((4, _CHUNK, _CH), jnp.float32),
            pltpu.VMEM_SHARED((npad, _CH), jnp.float32),
            pltpu.SemaphoreType.DMA,
            pltpu.SemaphoreType.DMA,
        ],
    )
    def k(m_hbm, xr_hbm, dst_hbm, out_hbm, idx_v, buf, acc_sh, lsem, ssem):
        t = lax.axis_index("s")
        pltpu.async_copy(
            xr_hbm.at[pl.ds(t * rpt, rpt), pl.ds(0, _CH)],
            acc_sh.at[pl.ds(t * rpt, rpt)], lsem)
        pltpu.sync_copy(dst_hbm.at[pl.ds(t * ke, ke)], idx_v)
        pltpu.make_async_copy(
            xr_hbm.at[pl.ds(t * rpt, rpt), pl.ds(0, _CH)],
            acc_sh.at[pl.ds(t * rpt, rpt)], lsem).wait()
        plsc.subcore_barrier()

        def load(j):
            pltpu.async_copy(
                m_hbm.at[pl.ds(t * ept + j * _CHUNK, _CHUNK),
                         pl.ds(0, _CH)], buf.at[lax.rem(j, 4)], lsem)

        def load_wait(j):
            pltpu.make_async_copy(
                m_hbm.at[pl.ds(0, _CHUNK), pl.ds(0, _CH)],
                buf.at[lax.rem(j, 4)], lsem).wait()

        def scat(j):
            pltpu.async_copy(buf.at[lax.rem(j, 4)], acc_sh.at[idx_v.at[j]],
                             ssem, add=True)

        def scat_wait(j):
            pltpu.make_async_copy(buf.at[lax.rem(j, 4)],
                                  acc_sh.at[idx_v.at[j]], ssem).wait()

        load(0)
        load(1)

        def step(j, c):
            @pl.when(j >= 2)
            def _():
                scat_wait(j - 2)
            load_wait(j)

            @pl.when(j + 2 < ke)
            def _():
                load(j + 2)
            scat(j)
            return c

        lax.fori_loop(0, ke, step, 0)
        scat_wait(ke - 2)
        scat_wait(ke - 1)
        plsc.subcore_barrier()
        pltpu.sync_copy(acc_sh.at[pl.ds(t * rpt, rpt)],
                        out_hbm.at[pl.ds(t * rpt, rpt), pl.ds(0, _CH)])

    return k(m128, xr128, dstE)


def _sc_pair(util, ia_flat, ib_flat):
    npad = util.shape[0]
    ppw = ia_flat.shape[0] // _NW
    mesh = plsc.VectorSubcoreMesh(core_axis_name="c", subcore_axis_name="s")

    @functools.partial(
        pl.kernel,
        out_type=jax.ShapeDtypeStruct((_NW * ppw,), jnp.float32),
        mesh=mesh,
        compiler_params=_SC_PARAMS,
        scratch_types=[
            pltpu.VMEM((npad,), jnp.float32),
            pltpu.VMEM((ppw,), jnp.int32),
            pltpu.VMEM((ppw,), jnp.int32),
            pltpu.VMEM((ppw,), jnp.float32),
        ],
    )
    def k(util_hbm, ia_hbm, ib_hbm, out_hbm, tab_v, ia_v, ib_v, o_v):
        w = lax.axis_index("s") * _NC + lax.axis_index("c")
        pltpu.sync_copy(util_hbm, tab_v)
        pltpu.sync_copy(ia_hbm.at[pl.ds(w * ppw, ppw)], ia_v)
        pltpu.sync_copy(ib_hbm.at[pl.ds(w * ppw, ppw)], ib_v)

        def body(j, c):
            s = pl.ds(j * 16, 16)
            va = plsc.load_gather(tab_v, [ia_v[s]])
            vb = plsc.load_gather(tab_v, [ib_v[s]])
            o_v[s] = vb - va
            return c

        lax.fori_loop(0, ppw // 16, body, 0)
        pltpu.sync_copy(o_v, out_hbm.at[pl.ds(w * ppw, ppw)])

    return k(util, ia_flat, ib_flat)


def _tc_dense(ep, msg128, w, bkm, xn128, root, bb, relu):
    ge = ep.shape[0] // _BE
    bn = _NPAD // ge

    def body(e_ref, g_ref, w_ref, bk_ref, xn_ref, root_ref, b_ref,
             m_ref, xr_ref):
        msgb = g_ref[:, :_CH]
        if relu:
            msgb = jnp.maximum(msgb, 0.0)
        eb = e_ref[...]
        o = (eb[:, :, None] * msgb[:, None, :]).reshape(_BE, _DE * _FIN)
        mm = (jnp.dot(o, w_ref[...], preferred_element_type=jnp.float32)
              + jnp.dot(msgb, bk_ref[...], preferred_element_type=jnp.float32))
        m_ref[...] = jnp.pad(mm, ((0, 0), (0, 128 - _CH)))
        xb = xn_ref[:, :_CH]
        if relu:
            xb = jnp.maximum(xb, 0.0)
        xr = jnp.dot(xb, root_ref[...],
                     preferred_element_type=jnp.float32) + b_ref[0:1, :]
        xr_ref[...] = jnp.pad(xr, ((0, 0), (0, 128 - _CH)))

    return pl.pallas_call(
        body,
        grid=(ge,),
        in_specs=[
            pl.BlockSpec((_BE, _DE), lambda g: (g, 0)),
            pl.BlockSpec((_BE, 128), lambda g: (g, 0)),
            pl.BlockSpec((_DE * _FIN, _CH), lambda g: (0, 0)),
            pl.BlockSpec((_FIN, _CH), lambda g: (0, 0)),
            pl.BlockSpec((bn, 128), lambda g: (g, 0)),
            pl.BlockSpec((_FIN, _CH), lambda g: (0, 0)),
            pl.BlockSpec((8, _CH), lambda g: (0, 0)),
        ],
        out_specs=[
            pl.BlockSpec((_BE, 128), lambda g: (g, 0)),
            pl.BlockSpec((bn, 128), lambda g: (g, 0)),
        ],
        out_shape=[
            jax.ShapeDtypeStruct((ep.shape[0], 128), jnp.float32),
            jax.ShapeDtypeStruct((_NPAD, 128), jnp.float32),
        ],
    )(ep, msg128, w, bkm, xn128, root, bb)


def _tc_util(ph128, wd, bdb):
    b = 1024
    g = _NPAD // b

    def body(h_ref, wd_ref, bd_ref, u_ref):
        h = jnp.maximum(h_ref[:, :_CH], 0.0)
        u_ref[...] = jnp.dot(h, wd_ref[...],
                             preferred_element_type=jnp.float32) + bd_ref[0:1, :]

    return pl.pallas_call(
        body,
        grid=(g,),
        in_specs=[
            pl.BlockSpec((b, 128), lambda i: (i, 0)),
            pl.BlockSpec((_CH, 1), lambda i: (0, 0)),
            pl.BlockSpec((8, 1), lambda i: (0, 0)),
        ],
        out_specs=pl.BlockSpec((b, 1), lambda i: (i, 0)),
        out_shape=jax.ShapeDtypeStruct((_NPAD, 1), jnp.float32),
    )(ph128, wd, bdb)


def kernel(x, a_edge_index, e, i, idx_a, idx_b,
           Wk1, bk1, root1, b1, Wk2, bk2, root2, b2, Wd, bd):
    x = x.astype(jnp.float32)
    e = e.astype(jnp.float32)
    src = a_edge_index[0].astype(jnp.int32)
    dst = a_edge_index[1].astype(jnp.int32)

    src128 = jnp.concatenate(
        [src, jnp.zeros((_EPAD - _E,), jnp.int32)]).reshape(-1, 128)
    dstE = jnp.concatenate(
        [dst, jnp.full((_EPAD - _E,), _DUMMY, jnp.int32)]).reshape(-1, 128)
    ep = jnp.pad(e, ((0, _EPAD - _E), (0, 0)))
    xp128 = jnp.pad(x, ((0, _NPAD - _N), (0, 128 - _FIN)))
    iaf = jnp.pad(idx_a.astype(jnp.int32), (0, _PPAD - _P))
    ibf = jnp.pad(idx_b.astype(jnp.int32), (0, _PPAD - _P))
    w1 = Wk1.reshape(_DE * _FIN, _CH)
    w2 = Wk2.reshape(_DE * _CH, _CH)
    bk1m = bk1.reshape(_FIN, _CH)
    bk2m = bk2.reshape(_CH, _CH)
    b1b = jnp.broadcast_to(b1[None, :], (8, _CH))
    b2b = jnp.broadcast_to(b2[None, :], (8, _CH))
    bdb = jnp.broadcast_to(bd[None, :], (8, 1))

    msg1 = _sc_gather(xp128, src128)
    m1, xr1 = _tc_dense(ep, msg1, w1, bk1m, xp128, root1, b1b, relu=False)
    ph = _sc_scatter(m1, xr1, dstE)

    msg2 = _sc_gather(ph, src128)
    m2, xr2 = _tc_dense(ep, msg2, w2, bk2m, ph, root2, b2b, relu=True)
    ph2 = _sc_scatter(m2, xr2, dstE)

    util = _tc_util(ph2, Wd, bdb)
    diff = _sc_pair(util.reshape(_NPAD), iaf, ibf)
    return diff[:_P, None]


# speedup vs baseline: 1.5188x; 1.5188x over previous
import functools

import jax
import jax.numpy as jnp
from jax import lax
from jax.experimental import pallas as pl
from jax.experimental.pallas import tpu as pltpu
import jax.experimental.pallas.tpu_sc as plsc

_E = 100000
_N = 50000
_P = 50000
_FIN = 32
_DE = 16
_CH = 32

_NC = 2
_NS = 16
_NW = _NC * _NS

_CHUNK = 128
_KG = 25
_EPAD = _NW * _KG * _CHUNK
_BE = 1024
_GE = _EPAD // _BE
_BN = 512
_NPAD = _GE * _BN
_KS = ((_EPAD + _NPAD) // _NS) // _CHUNK
_PPW = 1568
_PPAD = _NW * _PPW
_DUMMY = _N

_SC_PARAMS = pltpu.CompilerParams(use_tc_tiling_on_sc=False,
                                  needs_layout_passes=False)


def _sc_gather(table128, idx128):
    kg = idx128.shape[0] // _NW
    bpw = kg * _CHUNK
    mesh = plsc.VectorSubcoreMesh(core_axis_name="c", subcore_axis_name="s")

    @functools.partial(
        pl.kernel,
        out_type=jax.ShapeDtypeStruct((_NW * bpw, 128), jnp.float32),
        mesh=mesh,
        compiler_params=_SC_PARAMS,
        scratch_types=[
            pltpu.VMEM((kg, _CHUNK), jnp.int32),
            pltpu.VMEM((6, _CHUNK, 128), jnp.float32),
            pltpu.SemaphoreType.DMA,
            pltpu.SemaphoreType.DMA,
        ],
    )
    def k(table_hbm, idx_hbm, out_hbm, idx_v, fbuf, fsem, wsem):
        w = lax.axis_index("s") * _NC + lax.axis_index("c")
        pltpu.sync_copy(idx_hbm.at[pl.ds(w * kg, kg)], idx_v)

        def fetch(j):
            pltpu.async_copy(table_hbm.at[idx_v.at[j]],
                             fbuf.at[lax.rem(j, 6)], fsem)

        def fetch_wait(j):
            pltpu.make_async_copy(table_hbm.at[idx_v.at[j]],
                                  fbuf.at[lax.rem(j, 6)], fsem).wait()

        def write(j):
            pltpu.async_copy(
                fbuf.at[lax.rem(j, 6), pl.ds(0, _CHUNK), pl.ds(0, _CH)],
                out_hbm.at[pl.ds(w * bpw + j * _CHUNK, _CHUNK),
                           pl.ds(0, _CH)], wsem)

        def write_wait(j):
            pltpu.make_async_copy(
                fbuf.at[lax.rem(j, 6), pl.ds(0, _CHUNK), pl.ds(0, _CH)],
                out_hbm.at[pl.ds(w * bpw + j * _CHUNK, _CHUNK),
                           pl.ds(0, _CH)], wsem).wait()

        fetch(0)
        fetch(1)
        fetch(2)

        def step(j, c):
            @pl.when(j >= 3)
            def _():
                write_wait(j - 3)
            fetch_wait(j)

            @pl.when(j + 3 < kg)
            def _():
                fetch(j + 3)
            write(j)
            return c

        lax.fori_loop(0, kg, step, 0)
        write_wait(kg - 3)
        write_wait(kg - 2)
        write_wait(kg - 1)

    return k(table128, idx128)


def _sc_scatter(m128, xr128, dstE):
    ke = dstE.shape[0] // _NS
    npad = xr128.shape[0]
    rpt = npad // _NS
    ept = ke * _CHUNK
    mesh = plsc.VectorSubcoreMesh(core_axis_name="c", subcore_axis_name="s",
                                  num_cores=1)

    @functools.partial(
        pl.kernel,
        out_type=jax.ShapeDtypeStruct((npad, 128), jnp.float32),
        mesh=mesh,
        compiler_params=_SC_PARAMS,
        scratch_types=[
            pltpu.VMEM((ke, _CHUNK), jnp.int32),
            pltpu.VMEM((4, _CHUNK, _CH), jnp.float32),
            pltpu.VMEM_SHARED((npad, _CH), jnp.float32),
            pltpu.SemaphoreType.DMA,
            pltpu.SemaphoreType.DMA,
        ],
    )
    def k(m_hbm, xr_hbm, dst_hbm, out_hbm, idx_v, buf, acc_sh, lsem, ssem):
        t = lax.axis_index("s")
        pltpu.async_copy(
            xr_hbm.at[pl.ds(t * rpt, rpt), pl.ds(0, _CH)],
            acc_sh.at[pl.ds(t * rpt, rpt)], lsem)
        pltpu.sync_copy(dst_hbm.at[pl.ds(t * ke, ke)], idx_v)
        pltpu.make_async_copy(
            xr_hbm.at[pl.ds(t * rpt, rpt), pl.ds(0, _CH)],
            acc_sh.at[pl.ds(t * rpt, rpt)], lsem).wait()
        plsc.subcore_barrier()

        def load(j):
            pltpu.async_copy(
                m_hbm.at[pl.ds(t * ept + j * _CHUNK, _CHUNK),
                         pl.ds(0, _CH)], buf.at[lax.rem(j, 4)], lsem)

        def load_wait(j):
            pltpu.make_async_copy(
                m_hbm.at[pl.ds(0, _CHUNK), pl.ds(0, _CH)],
                buf.at[lax.rem(j, 4)], lsem).wait()

        def scat(j):
            pltpu.async_copy(buf.at[lax.rem(j, 4)], acc_sh.at[idx_v.at[j]],
                             ssem, add=True)

        def scat_wait(j):
            pltpu.make_async_copy(buf.at[lax.rem(j, 4)],
                                  acc_sh.at[idx_v.at[j]], ssem).wait()

        load(0)
        load(1)

        def step(j, c):
            @pl.when(j >= 2)
            def _():
                scat_wait(j - 2)
            load_wait(j)

            @pl.when(j + 2 < ke)
            def _():
                load(j + 2)
            scat(j)
            return c

        lax.fori_loop(0, ke, step, 0)
        scat_wait(ke - 2)
        scat_wait(ke - 1)
        plsc.subcore_barrier()
        pltpu.sync_copy(acc_sh.at[pl.ds(t * rpt, rpt)],
                        out_hbm.at[pl.ds(t * rpt, rpt), pl.ds(0, _CH)])

    return k(m128, xr128, dstE)


def _sc_pair(util, ia_flat, ib_flat):
    npad = util.shape[0]
    ppw = ia_flat.shape[0] // _NW
    mesh = plsc.VectorSubcoreMesh(core_axis_name="c", subcore_axis_name="s")

    @functools.partial(
        pl.kernel,
        out_type=jax.ShapeDtypeStruct((_NW * ppw,), jnp.float32),
        mesh=mesh,
        compiler_params=_SC_PARAMS,
        scratch_types=[
            pltpu.VMEM((npad,), jnp.float32),
            pltpu.VMEM((ppw,), jnp.int32),
            pltpu.VMEM((ppw,), jnp.int32),
            pltpu.VMEM((ppw,), jnp.float32),
        ],
    )
    def k(util_hbm, ia_hbm, ib_hbm, out_hbm, tab_v, ia_v, ib_v, o_v):
        w = lax.axis_index("s") * _NC + lax.axis_index("c")
        pltpu.sync_copy(util_hbm, tab_v)
        pltpu.sync_copy(ia_hbm.at[pl.ds(w * ppw, ppw)], ia_v)
        pltpu.sync_copy(ib_hbm.at[pl.ds(w * ppw, ppw)], ib_v)

        def body(j, c):
            s = pl.ds(j * 16, 16)
            va = plsc.load_gather(tab_v, [ia_v[s]])
            vb = plsc.load_gather(tab_v, [ib_v[s]])
            o_v[s] = vb - va
            return c

        lax.fori_loop(0, ppw // 16, body, 0)
        pltpu.sync_copy(o_v, out_hbm.at[pl.ds(w * ppw, ppw)])

    return k(util, ia_flat, ib_flat)


def _tc_dense(ep, msg128, w, bkm, rmat, tmat, xn128, root, bb, relu):
    ge = ep.shape[0] // _BE
    bn = _NPAD // ge

    def body(e_ref, g_ref, w_ref, bk_ref, r_ref, t_ref, xn_ref, root_ref,
             b_ref, m_ref, xr_ref):
        msgb = g_ref[:, :_CH]
        if relu:
            msgb = jnp.maximum(msgb, 0.0)
        eb = e_ref[...]
        e_exp = jnp.dot(eb, r_ref[...], preferred_element_type=jnp.float32)
        msg_t = jnp.dot(msgb, t_ref[...], preferred_element_type=jnp.float32)
        o = e_exp * msg_t
        mm = (jnp.dot(o, w_ref[...], preferred_element_type=jnp.float32)
              + jnp.dot(msgb, bk_ref[...], preferred_element_type=jnp.float32))
        m_ref[...] = jnp.pad(mm, ((0, 0), (0, 128 - _CH)))
        xb = xn_ref[:, :_CH]
        if relu:
            xb = jnp.maximum(xb, 0.0)
        xr = jnp.dot(xb, root_ref[...],
                     preferred_element_type=jnp.float32) + b_ref[0:1, :]
        xr_ref[...] = jnp.pad(xr, ((0, 0), (0, 128 - _CH)))

    return pl.pallas_call(
        body,
        grid=(ge,),
        in_specs=[
            pl.BlockSpec((_BE, _DE), lambda g: (g, 0)),
            pl.BlockSpec((_BE, 128), lambda g: (g, 0)),
            pl.BlockSpec((_DE * _FIN, _CH), lambda g: (0, 0)),
            pl.BlockSpec((_FIN, _CH), lambda g: (0, 0)),
            pl.BlockSpec((_DE, _DE * _FIN), lambda g: (0, 0)),
            pl.BlockSpec((_FIN, _DE * _FIN), lambda g: (0, 0)),
            pl.BlockSpec((bn, 128), lambda g: (g, 0)),
            pl.BlockSpec((_FIN, _CH), lambda g: (0, 0)),
            pl.BlockSpec((8, _CH), lambda g: (0, 0)),
        ],
        out_specs=[
            pl.BlockSpec((_BE, 128), lambda g: (g, 0)),
            pl.BlockSpec((bn, 128), lambda g: (g, 0)),
        ],
        out_shape=[
            jax.ShapeDtypeStruct((ep.shape[0], 128), jnp.float32),
            jax.ShapeDtypeStruct((_NPAD, 128), jnp.float32),
        ],
    )(ep, msg128, w, bkm, rmat, tmat, xn128, root, bb)


def _tc_util(ph128, wd, bdb):
    b = 1024
    g = _NPAD // b

    def body(h_ref, wd_ref, bd_ref, u_ref):
        h = jnp.maximum(h_ref[:, :_CH], 0.0)
        u_ref[...] = jnp.dot(h, wd_ref[...],
                             preferred_element_type=jnp.float32) + bd_ref[0:1, :]

    return pl.pallas_call(
        body,
        grid=(g,),
        in_specs=[
            pl.BlockSpec((b, 128), lambda i: (i, 0)),
            pl.BlockSpec((_CH, 1), lambda i: (0, 0)),
            pl.BlockSpec((8, 1), lambda i: (0, 0)),
        ],
        out_specs=pl.BlockSpec((b, 1), lambda i: (i, 0)),
        out_shape=jax.ShapeDtypeStruct((_NPAD, 1), jnp.float32),
    )(ph128, wd, bdb)


def kernel(x, a_edge_index, e, i, idx_a, idx_b,
           Wk1, bk1, root1, b1, Wk2, bk2, root2, b2, Wd, bd):
    x = x.astype(jnp.float32)
    e = e.astype(jnp.float32)
    src = a_edge_index[0].astype(jnp.int32)
    dst = a_edge_index[1].astype(jnp.int32)

    src128 = jnp.concatenate(
        [src, jnp.zeros((_EPAD - _E,), jnp.int32)]).reshape(-1, 128)
    dstE = jnp.concatenate(
        [dst, jnp.full((_EPAD - _E,), _DUMMY, jnp.int32)]).reshape(-1, 128)
    ep = jnp.pad(e, ((0, _EPAD - _E), (0, 0)))
    xp128 = jnp.pad(x, ((0, _NPAD - _N), (0, 128 - _FIN)))
    iaf = jnp.pad(idx_a.astype(jnp.int32), (0, _PPAD - _P))
    ibf = jnp.pad(idx_b.astype(jnp.int32), (0, _PPAD - _P))
    w1 = Wk1.reshape(_DE * _FIN, _CH)
    w2 = Wk2.reshape(_DE * _CH, _CH)
    bk1m = bk1.reshape(_FIN, _CH)
    bk2m = bk2.reshape(_CH, _CH)
    rmat = jnp.repeat(jnp.eye(_DE, dtype=jnp.float32), _FIN, axis=1)
    tmat = jnp.tile(jnp.eye(_FIN, dtype=jnp.float32), (1, _DE))
    b1b = jnp.broadcast_to(b1[None, :], (8, _CH))
    b2b = jnp.broadcast_to(b2[None, :], (8, _CH))
    bdb = jnp.broadcast_to(bd[None, :], (8, 1))

    msg1 = _sc_gather(xp128, src128)
    m1, xr1 = _tc_dense(ep, msg1, w1, bk1m, rmat, tmat, xp128,
                        root1, b1b, relu=False)
    ph = _sc_scatter(m1, xr1, dstE)

    msg2 = _sc_gather(ph, src128)
    m2, xr2 = _tc_dense(ep, msg2, w2, bk2m, rmat, tmat, ph,
                        root2, b2b, relu=True)
    ph2 = _sc_scatter(m2, xr2, dstE)

    util = _tc_util(ph2, Wd, bdb)
    diff = _sc_pair(util.reshape(_NPAD), iaf, ibf)
    return diff[:_P, None]
